# baseline (speedup 1.0000x reference)
import jax
import jax.numpy as jnp
from jax.experimental import pallas as pl
from jax.experimental.pallas import tpu as pltpu

B = 4096
I = 4096
O = 4096
NCORES = 2
BR = 256
KW = (O // NCORES) // BR
KX = (B // NCORES) // BR


def _wsum_body(w_ref, o_ref):
    k = pl.program_id(1)
    part = jnp.sum(w_ref[...], axis=0, keepdims=True)

    @pl.when(k == 0)
    def _init():
        o_ref[...] = part[None]

    @pl.when(k > 0)
    def _acc():
        o_ref[...] += part[None]


def _out_body(x_ref, ws_ref, b_ref, o_ref):
    wsum = jnp.sum(ws_ref[...], axis=0)
    part = jnp.sum(x_ref[...] * wsum, axis=1, keepdims=True)
    o_ref[...] = part + jnp.sum(b_ref[...])


def kernel(x, W, b):
    wpart = pl.pallas_call(
        _wsum_body,
        grid=(NCORES, KW),
        in_specs=[pl.BlockSpec((BR, I), lambda c, k: (c * KW + k, 0))],
        out_specs=pl.BlockSpec((1, 1, I), lambda c, k: (c, 0, 0)),
        out_shape=jax.ShapeDtypeStruct((NCORES, 1, I), jnp.float32),
        compiler_params=pltpu.CompilerParams(
            dimension_semantics=("parallel", "arbitrary"),
        ),
    )(W)

    out = pl.pallas_call(
        _out_body,
        grid=(NCORES, KX),
        in_specs=[
            pl.BlockSpec((BR, I), lambda c, k: (c * KX + k, 0)),
            pl.BlockSpec((NCORES, 1, I), lambda c, k: (0, 0, 0)),
            pl.BlockSpec((1, I), lambda c, k: (0, 0)),
        ],
        out_specs=pl.BlockSpec((BR, 1), lambda c, k: (c * KX + k, 0)),
        out_shape=jax.ShapeDtypeStruct((B, 1), jnp.float32),
        compiler_params=pltpu.CompilerParams(
            dimension_semantics=("parallel", "arbitrary"),
        ),
    )(x, wpart, b.reshape(1, I))
    return out


# speedup vs baseline: 1.0666x; 1.0666x over previous
import jax
import jax.numpy as jnp
from jax.experimental import pallas as pl
from jax.experimental.pallas import tpu as pltpu

B = 4096
I = 4096
O = 4096
CH = 512
NW = O // CH
NX = B // CH
NBUF = 4


def _body(x_hbm, w_hbm, b_ref, o_ref, buf, ws_ref, sems):
    def copy(i):
        if i < NW:
            src = w_hbm.at[pl.ds(i * CH, CH), :]
        else:
            src = x_hbm.at[pl.ds((i - NW) * CH, CH), :]
        return pltpu.make_async_copy(src, buf.at[i % NBUF], sems.at[i % NBUF])

    for i in range(NBUF):
        copy(i).start()

    bsum = jnp.sum(b_ref[...])

    for i in range(NW + NX):
        copy(i).wait()
        data = buf[i % NBUF]
        if i == 0:
            ws_ref[...] = jnp.sum(data, axis=0, keepdims=True)
        elif i < NW:
            ws_ref[...] += jnp.sum(data, axis=0, keepdims=True)
        else:
            part = jnp.sum(data * ws_ref[...], axis=1, keepdims=True)
            o_ref[pl.ds((i - NW) * CH, CH), :] = part + bsum
        if i + NBUF < NW + NX:
            copy(i + NBUF).start()


def kernel(x, W, b):
    return pl.pallas_call(
        _body,
        in_specs=[
            pl.BlockSpec(memory_space=pltpu.MemorySpace.HBM),
            pl.BlockSpec(memory_space=pltpu.MemorySpace.HBM),
            pl.BlockSpec((1, I), lambda: (0, 0)),
        ],
        out_specs=pl.BlockSpec((B, 1), lambda: (0, 0)),
        out_shape=jax.ShapeDtypeStruct((B, 1), jnp.float32),
        scratch_shapes=[
            pltpu.VMEM((NBUF, CH, I), jnp.float32),
            pltpu.VMEM((1, I), jnp.float32),
            pltpu.SemaphoreType.DMA((NBUF,)),
        ],
    )(x, W, b.reshape(1, I))


# speedup vs baseline: 1.0852x; 1.0174x over previous
import jax
import jax.numpy as jnp
from jax.experimental import pallas as pl
from jax.experimental.pallas import tpu as pltpu

B = 4096
I = 4096
O = 4096
CH = 256
NW = O // CH
NX = B // CH
NBUF = 8


def _body(x_hbm, w_hbm, b_ref, o_ref, buf, ws_ref, sems):
    def copy(i):
        if i < NW:
            src = w_hbm.at[pl.ds(i * CH, CH), :]
        else:
            src = x_hbm.at[pl.ds((i - NW) * CH, CH), :]
        return pltpu.make_async_copy(src, buf.at[i % NBUF], sems.at[i % NBUF])

    for i in range(NBUF):
        copy(i).start()

    bsum = jnp.sum(b_ref[...])

    for i in range(NW + NX):
        copy(i).wait()
        data = buf[i % NBUF]
        if i == 0:
            ws_ref[...] = jnp.sum(data, axis=0, keepdims=True)
        elif i < NW:
            ws_ref[...] += jnp.sum(data, axis=0, keepdims=True)
        else:
            part = jnp.sum(data * ws_ref[...], axis=1, keepdims=True)
            o_ref[pl.ds((i - NW) * CH, CH), :] = part + bsum
        if i + NBUF < NW + NX:
            copy(i + NBUF).start()


def kernel(x, W, b):
    return pl.pallas_call(
        _body,
        in_specs=[
            pl.BlockSpec(memory_space=pltpu.MemorySpace.HBM),
            pl.BlockSpec(memory_space=pltpu.MemorySpace.HBM),
            pl.BlockSpec((1, I), lambda: (0, 0)),
        ],
        out_specs=pl.BlockSpec((B, 1), lambda: (0, 0)),
        out_shape=jax.ShapeDtypeStruct((B, 1), jnp.float32),
        scratch_shapes=[
            pltpu.VMEM((NBUF, CH, I), jnp.float32),
            pltpu.VMEM((1, I), jnp.float32),
            pltpu.SemaphoreType.DMA((NBUF,)),
        ],
    )(x, W, b.reshape(1, I))


# speedup vs baseline: 1.0940x; 1.0081x over previous
import jax
import jax.numpy as jnp
from jax.experimental import pallas as pl
from jax.experimental.pallas import tpu as pltpu

B = 4096
I = 4096
O = 4096
CH = 128
NW = O // CH
NX = B // CH
NBUF = 16


def _body(x_hbm, w_hbm, b_ref, o_ref, buf, ws_ref, sems):
    def copy(i):
        if i < NW:
            src = w_hbm.at[pl.ds(i * CH, CH), :]
        else:
            src = x_hbm.at[pl.ds((i - NW) * CH, CH), :]
        return pltpu.make_async_copy(src, buf.at[i % NBUF], sems.at[i % NBUF])

    for i in range(NBUF):
        copy(i).start()

    bsum = jnp.sum(b_ref[...])

    for i in range(NW + NX):
        copy(i).wait()
        data = buf[i % NBUF]
        if i == 0:
            ws_ref[...] = jnp.sum(data, axis=0, keepdims=True)
        elif i < NW:
            ws_ref[...] += jnp.sum(data, axis=0, keepdims=True)
        else:
            part = jnp.sum(data * ws_ref[...], axis=1, keepdims=True)
            o_ref[pl.ds((i - NW) * CH, CH), :] = part + bsum
        if i + NBUF < NW + NX:
            copy(i + NBUF).start()


def kernel(x, W, b):
    return pl.pallas_call(
        _body,
        in_specs=[
            pl.BlockSpec(memory_space=pltpu.MemorySpace.HBM),
            pl.BlockSpec(memory_space=pltpu.MemorySpace.HBM),
            pl.BlockSpec((1, I), lambda: (0, 0)),
        ],
        out_specs=pl.BlockSpec((B, 1), lambda: (0, 0)),
        out_shape=jax.ShapeDtypeStruct((B, 1), jnp.float32),
        scratch_shapes=[
            pltpu.VMEM((NBUF, CH, I), jnp.float32),
            pltpu.VMEM((1, I), jnp.float32),
            pltpu.SemaphoreType.DMA((NBUF,)),
        ],
    )(x, W, b.reshape(1, I))
